# baseline (speedup 1.0000x reference)
import functools

import jax
import jax.numpy as jnp
from jax import lax
from jax.experimental import pallas as pl
from jax.experimental.pallas import tpu as pltpu
from jax.experimental.pallas import tpu_sc as plsc

N = 5000
L = 16
NW = 32
NPS = 5008
NPT = 5120
TPW = NPT // NW
NCH = NPS // L


def _bcast16(vec, lane):
    idx = jnp.full((L, 1), lane, jnp.int32)
    dnums = lax.GatherDimensionNumbers(
        offset_dims=(), collapsed_slice_dims=(0,), start_index_map=(0,))
    return lax.gather(vec, idx, dnums, slice_sizes=(1,),
                      mode=lax.GatherScatterMode.PROMISE_IN_BOUNDS)


def _sc_kernel(tx1h, ty1h, tx2h, ty2h, sx1h, sy1h, sx2h, sy2h,
               tot_h, cnt_h,
               tx1v, ty1v, tx2v, ty2v, sx1v, sy1v, sx2v, sy2v, sav,
               obuf):
    wid = lax.axis_index("s") * 2 + lax.axis_index("c")
    base = wid * TPW

    pltpu.sync_copy(tx1h.at[pl.ds(base, TPW)], tx1v)
    pltpu.sync_copy(ty1h.at[pl.ds(base, TPW)], ty1v)
    pltpu.sync_copy(tx2h.at[pl.ds(base, TPW)], tx2v)
    pltpu.sync_copy(ty2h.at[pl.ds(base, TPW)], ty2v)
    pltpu.sync_copy(sx1h, sx1v)
    pltpu.sync_copy(sy1h, sy1v)
    pltpu.sync_copy(sx2h, sx2v)
    pltpu.sync_copy(sy2h, sy2v)

    def _area(c, _):
        sl = pl.ds(c * L, L)
        sav[sl] = (sx2v[sl] - sx1v[sl]) * (sy2v[sl] - sy1v[sl])
        return _
    lax.fori_loop(0, NCH, _area, 0, unroll=4)

    def _teacher_chunk(tc, carry):
        acc_t, acc_c = carry
        tsl = pl.ds(tc * L, L)
        tvx1 = tx1v[tsl]
        tvy1 = ty1v[tsl]
        tvx2 = tx2v[tsl]
        tvy2 = ty2v[tsl]
        tva = (tvx2 - tvx1) * (tvy2 - tvy1)
        for lane in range(L):
            tx1 = _bcast16(tvx1, lane)
            ty1 = _bcast16(tvy1, lane)
            tx2 = _bcast16(tvx2, lane)
            ty2 = _bcast16(tvy2, lane)
            tarea = _bcast16(tva, lane)

            def _chunk(c, inner, tx1=tx1, ty1=ty1, tx2=tx2, ty2=ty2,
                       tarea=tarea):
                a_t, a_c = inner
                sl = pl.ds(c * L, L)
                sx1 = sx1v[sl]
                sy1 = sy1v[sl]
                sx2 = sx2v[sl]
                sy2 = sy2v[sl]
                iw = jnp.maximum(
                    jnp.minimum(tx2, sx2) - jnp.maximum(tx1, sx1), 0.0)
                ih = jnp.maximum(
                    jnp.minimum(ty2, sy2) - jnp.maximum(ty1, sy1), 0.0)
                inter = iw * ih
                union = (tarea + sav[sl]) - inter
                d0 = tx1 - sx1
                d1 = ty1 - sy1
                d2 = tx2 - sx2
                d3 = ty2 - sy2
                sq = d0 * d0 + d1 * d1 + d2 * d2 + d3 * d3
                g = (inter + inter) - union
                mf = jnp.minimum(jnp.maximum(g * 1e30, 0.0), 1.0)
                mf = mf * jnp.minimum(jnp.maximum(union * 1e30, 0.0), 1.0)
                a_t = a_t + mf * sq
                a_c = a_c + mf
                return a_t, a_c

            acc_t, acc_c = lax.fori_loop(0, NCH, _chunk, (acc_t, acc_c))
        return acc_t, acc_c

    zero = jnp.zeros((L,), jnp.float32)
    acc_t, acc_c = lax.fori_loop(0, TPW // L, _teacher_chunk, (zero, zero))

    obuf[...] = acc_t
    pltpu.sync_copy(obuf, tot_h.at[wid])
    obuf[...] = acc_c
    pltpu.sync_copy(obuf, cnt_h.at[wid])


@functools.partial(
    pl.kernel,
    out_type=(
        jax.ShapeDtypeStruct((NW, L), jnp.float32),
        jax.ShapeDtypeStruct((NW, L), jnp.float32),
    ),
    mesh=plsc.VectorSubcoreMesh(core_axis_name="c", subcore_axis_name="s"),
    scratch_types=[
        pltpu.VMEM((TPW,), jnp.float32),
        pltpu.VMEM((TPW,), jnp.float32),
        pltpu.VMEM((TPW,), jnp.float32),
        pltpu.VMEM((TPW,), jnp.float32),
        pltpu.VMEM((NPS,), jnp.float32),
        pltpu.VMEM((NPS,), jnp.float32),
        pltpu.VMEM((NPS,), jnp.float32),
        pltpu.VMEM((NPS,), jnp.float32),
        pltpu.VMEM((NPS,), jnp.float32),
        pltpu.VMEM((L,), jnp.float32),
    ],
)
def _sc_call(*refs):
    _sc_kernel(*refs)


def _finish_kernel(tot_ref, cnt_ref, out_ref):
    tot = jnp.sum(tot_ref[...])
    cnt = jnp.sum(cnt_ref[...])
    out_ref[...] = jnp.where(
        cnt > 0.0,
        tot / (4.0 * jnp.maximum(cnt, 1.0)),
        0.0,
    )[None, None]


def _pad_col(col, npad, sentinel):
    return jnp.concatenate(
        [col.astype(jnp.float32), jnp.full((npad,), sentinel, jnp.float32)]
    )


def kernel(teacher_boxes, teacher_scores, student_boxes, student_scores):
    del teacher_scores, student_scores
    tb = teacher_boxes.astype(jnp.float32)
    sb = student_boxes.astype(jnp.float32)
    tcols = [_pad_col(tb[:, k], NPT - N, [-8.0, -8.0, -7.0, -7.0][k])
             for k in range(4)]
    scols = [_pad_col(sb[:, k], NPS - N, [8.0, 8.0, 9.0, 9.0][k])
             for k in range(4)]

    tot_p, cnt_p = _sc_call(*tcols, *scols)

    out = pl.pallas_call(
        _finish_kernel,
        out_shape=jax.ShapeDtypeStruct((1, 1), jnp.float32),
    )(tot_p, cnt_p)
    return out[0, 0]


# speedup vs baseline: 3.4399x; 3.4399x over previous
import functools

import jax
import jax.numpy as jnp
from jax import lax
from jax.experimental import pallas as pl
from jax.experimental.pallas import tpu as pltpu
from jax.experimental.pallas import tpu_sc as plsc

N = 5000
L = 16
NW = 32
NPS = 5008
NPT = 5120
TPW = 32
SC_ROWS = NW * TPW
NCH = NPS // L

TBLK = 512
SBLK = 1280
TC_TILE0 = SC_ROWS // TBLK


def _bcast16(vec, lane):
    idx = jnp.full((L, 1), lane, jnp.int32)
    dnums = lax.GatherDimensionNumbers(
        offset_dims=(), collapsed_slice_dims=(0,), start_index_map=(0,))
    return lax.gather(vec, idx, dnums, slice_sizes=(1,),
                      mode=lax.GatherScatterMode.PROMISE_IN_BOUNDS)


def _sc_kernel(tx1h, ty1h, tx2h, ty2h, sx1h, sy1h, sx2h, sy2h,
               tot_h, cnt_h,
               tx1v, ty1v, tx2v, ty2v, sx1v, sy1v, sx2v, sy2v, sav,
               obuf):
    wid = lax.axis_index("s") * 2 + lax.axis_index("c")
    base = wid * TPW

    pltpu.sync_copy(tx1h.at[pl.ds(base, TPW)], tx1v)
    pltpu.sync_copy(ty1h.at[pl.ds(base, TPW)], ty1v)
    pltpu.sync_copy(tx2h.at[pl.ds(base, TPW)], tx2v)
    pltpu.sync_copy(ty2h.at[pl.ds(base, TPW)], ty2v)
    pltpu.sync_copy(sx1h, sx1v)
    pltpu.sync_copy(sy1h, sy1v)
    pltpu.sync_copy(sx2h, sx2v)
    pltpu.sync_copy(sy2h, sy2v)

    def _area(c, _):
        sl = pl.ds(c * L, L)
        sav[sl] = (sx2v[sl] - sx1v[sl]) * (sy2v[sl] - sy1v[sl])
        return _
    lax.fori_loop(0, NCH, _area, 0, unroll=4)

    def _teacher_chunk(tc, carry):
        acc_t, acc_c = carry
        tsl = pl.ds(tc * L, L)
        tvx1 = tx1v[tsl]
        tvy1 = ty1v[tsl]
        tvx2 = tx2v[tsl]
        tvy2 = ty2v[tsl]
        tva = (tvx2 - tvx1) * (tvy2 - tvy1)
        for lane in range(L):
            tx1 = _bcast16(tvx1, lane)
            ty1 = _bcast16(tvy1, lane)
            tx2 = _bcast16(tvx2, lane)
            ty2 = _bcast16(tvy2, lane)
            tarea = _bcast16(tva, lane)

            def _chunk(c, inner, tx1=tx1, ty1=ty1, tx2=tx2, ty2=ty2,
                       tarea=tarea):
                a_t, a_c = inner
                sl = pl.ds(c * L, L)
                sx1 = sx1v[sl]
                sy1 = sy1v[sl]
                sx2 = sx2v[sl]
                sy2 = sy2v[sl]
                iw = jnp.maximum(
                    jnp.minimum(tx2, sx2) - jnp.maximum(tx1, sx1), 0.0)
                ih = jnp.maximum(
                    jnp.minimum(ty2, sy2) - jnp.maximum(ty1, sy1), 0.0)
                inter = iw * ih
                union = (tarea + sav[sl]) - inter
                d0 = tx1 - sx1
                d1 = ty1 - sy1
                d2 = tx2 - sx2
                d3 = ty2 - sy2
                sq = d0 * d0 + d1 * d1 + d2 * d2 + d3 * d3
                g = (inter + inter) - union
                mf = jnp.minimum(jnp.maximum(g * 1e30, 0.0), 1.0)
                mf = mf * jnp.minimum(jnp.maximum(union * 1e30, 0.0), 1.0)
                a_t = a_t + mf * sq
                a_c = a_c + mf
                return a_t, a_c

            acc_t, acc_c = lax.fori_loop(0, NCH, _chunk, (acc_t, acc_c))
        return acc_t, acc_c

    zero = jnp.zeros((L,), jnp.float32)
    acc_t, acc_c = lax.fori_loop(0, TPW // L, _teacher_chunk, (zero, zero))

    obuf[...] = acc_t
    pltpu.sync_copy(obuf, tot_h.at[wid])
    obuf[...] = acc_c
    pltpu.sync_copy(obuf, cnt_h.at[wid])


@functools.partial(
    pl.kernel,
    out_type=(
        jax.ShapeDtypeStruct((NW, L), jnp.float32),
        jax.ShapeDtypeStruct((NW, L), jnp.float32),
    ),
    mesh=plsc.VectorSubcoreMesh(core_axis_name="c", subcore_axis_name="s"),
    scratch_types=[
        pltpu.VMEM((TPW,), jnp.float32),
        pltpu.VMEM((TPW,), jnp.float32),
        pltpu.VMEM((TPW,), jnp.float32),
        pltpu.VMEM((TPW,), jnp.float32),
        pltpu.VMEM((NPS,), jnp.float32),
        pltpu.VMEM((NPS,), jnp.float32),
        pltpu.VMEM((NPS,), jnp.float32),
        pltpu.VMEM((NPS,), jnp.float32),
        pltpu.VMEM((NPS,), jnp.float32),
        pltpu.VMEM((L,), jnp.float32),
    ],
)
def _sc_call(*refs):
    _sc_kernel(*refs)


def _tc_tile_kernel(t_ref, s_ref, tot_ref, cnt_ref):
    i = pl.program_id(0)
    j = pl.program_id(1)

    @pl.when((i == 0) & (j == 0))
    def _init():
        tot_ref[...] = jnp.zeros_like(tot_ref)
        cnt_ref[...] = jnp.zeros_like(cnt_ref)

    t = t_ref[...]
    s = s_ref[...]

    tx1 = t[:, 0:1]
    ty1 = t[:, 1:2]
    tx2 = t[:, 2:3]
    ty2 = t[:, 3:4]
    sx1 = s[0:1, :]
    sy1 = s[1:2, :]
    sx2 = s[2:3, :]
    sy2 = s[3:4, :]

    iw = jnp.maximum(jnp.minimum(tx2, sx2) - jnp.maximum(tx1, sx1), 0.0)
    ih = jnp.maximum(jnp.minimum(ty2, sy2) - jnp.maximum(ty1, sy1), 0.0)
    inter = iw * ih

    tarea = (tx2 - tx1) * (ty2 - ty1)
    sarea = (sx2 - sx1) * (sy2 - sy1)
    union = (tarea + sarea) - inter

    mask = ((inter + inter) >= union) & (union > 0.0)

    d0 = tx1 - sx1
    d1 = ty1 - sy1
    d2 = tx2 - sx2
    d3 = ty2 - sy2
    sq = d0 * d0 + d1 * d1 + d2 * d2 + d3 * d3

    tot_ref[...] += jnp.sum(jnp.where(mask, sq, 0.0), keepdims=True)
    cnt_ref[...] += jnp.sum(mask.astype(jnp.float32), keepdims=True)


def _combine_kernel(sc_tot_ref, sc_cnt_ref, tc_tot_ref, tc_cnt_ref, out_ref):
    tot = jnp.sum(sc_tot_ref[...]) + tc_tot_ref[0, 0]
    cnt = jnp.sum(sc_cnt_ref[...]) + tc_cnt_ref[0, 0]
    out_ref[...] = jnp.where(
        cnt > 0.0,
        tot / (4.0 * jnp.maximum(cnt, 1.0)),
        0.0,
    )[None, None]


def _pad_col(col, npad, sentinel):
    return jnp.concatenate(
        [col.astype(jnp.float32), jnp.full((npad,), sentinel, jnp.float32)]
    )


def kernel(teacher_boxes, teacher_scores, student_boxes, student_scores):
    del teacher_scores, student_scores
    tb = teacher_boxes.astype(jnp.float32)
    sb = student_boxes.astype(jnp.float32)
    tsent = [-8.0, -8.0, -7.0, -7.0]
    ssent = [8.0, 8.0, 9.0, 9.0]
    tcols = [_pad_col(tb[:, k], NPT - N, tsent[k]) for k in range(4)]
    scols = [_pad_col(sb[:, k], NPS - N, ssent[k]) for k in range(4)]

    t2d = jnp.concatenate(
        [tb, jnp.tile(jnp.asarray(tsent, jnp.float32)[None, :], (NPT - N, 1))]
    )
    s2d = jnp.concatenate(
        [sb, jnp.tile(jnp.asarray(ssent, jnp.float32)[None, :], (NPT - N, 1))]
    ).T

    sc_tot, sc_cnt = _sc_call(*tcols, *scols)

    tc_tot, tc_cnt = pl.pallas_call(
        _tc_tile_kernel,
        grid=((NPT - SC_ROWS) // TBLK, NPT // SBLK),
        in_specs=[
            pl.BlockSpec((TBLK, 4), lambda i, j: (i + TC_TILE0, 0)),
            pl.BlockSpec((4, SBLK), lambda i, j: (0, j)),
        ],
        out_specs=[
            pl.BlockSpec((1, 1), lambda i, j: (0, 0)),
            pl.BlockSpec((1, 1), lambda i, j: (0, 0)),
        ],
        out_shape=[
            jax.ShapeDtypeStruct((1, 1), jnp.float32),
            jax.ShapeDtypeStruct((1, 1), jnp.float32),
        ],
    )(t2d, s2d)

    out = pl.pallas_call(
        _combine_kernel,
        out_shape=jax.ShapeDtypeStruct((1, 1), jnp.float32),
    )(sc_tot, sc_cnt, tc_tot, tc_cnt)
    return out[0, 0]
